# Initial kernel scaffold; baseline (speedup 1.0000x reference)
import jax
import jax.numpy as jnp
from jax.experimental import pallas as pl


def kernel(x_seq, edge_index, l1_wl, l1_bl, l1_wr, ln1_g, ln1_b, l2_wl, l2_bl, l2_wr, ln2_g, ln2_b, gru_w_ih, gru_w_hh, gru_b_ih, gru_b_hh, head_w, head_b):
    raise NotImplementedError("write your pallas kernel here")


# speedup vs baseline: 31.2537x; 31.2537x over previous
import functools

import jax
import jax.numpy as jnp
from jax import lax
from jax.experimental import pallas as pl
from jax.experimental.pallas import tpu as pltpu
from jax.experimental.pallas import tpu_sc as plsc

N = 10000
NPAD = 10240
E = 160000
B, T = 4, 12
BT = B * T
HG = 64
HT = 128
NC, NS = 2, 16
RPT = NPAD // NS
NB1 = (E // (NC * NS) + 127) // 128
NB2 = (E // NS + 127) // 128
G = 24
GPC = G // NC
DUMMY = N


def _sc_mesh():
    return plsc.VectorSubcoreMesh(
        core_axis_name="c", subcore_axis_name="s",
        num_cores=NC, num_subcores=NS)


def _agg1_body(x0_hbm, src_hbm, dst_hbm, z_hbm, out_hbm, src_v, dst_v, buf_v,
               acc_sh):
    c = lax.axis_index("c")
    s = lax.axis_index("s")
    pltpu.sync_copy(src_hbm.at[c, s], src_v)
    pltpu.sync_copy(dst_hbm.at[c, s], dst_v)
    pltpu.sync_copy(z_hbm, acc_sh.at[pl.ds(s * RPT, RPT)])
    plsc.subcore_barrier()

    def body(j, _):
        pltpu.sync_copy(x0_hbm.at[src_v.at[j]], buf_v)
        pltpu.sync_copy(buf_v, acc_sh.at[dst_v.at[j]], add=True)
        return 0
    lax.fori_loop(0, NB1, body, 0)
    plsc.subcore_barrier()
    pltpu.sync_copy(acc_sh.at[pl.ds(s * RPT, RPT)],
                    out_hbm.at[c, pl.ds(s * RPT, RPT)])


def _agg2_body(x1_hbm, src_hbm, dst_hbm, z_hbm, out_hbm, src_v, dst_v, buf_v,
               acc_sh):
    c = lax.axis_index("c")
    s = lax.axis_index("s")
    pltpu.sync_copy(dst_hbm.at[s], dst_v)

    def panel(gl, _):
        g = c * GPC + gl
        pltpu.sync_copy(src_hbm.at[s], src_v)

        def scale(j, _):
            for cch in range(8):
                sl = pl.ds(cch * 16, 16)
                src_v[j, sl] = src_v[j, sl] * G + g
            return 0
        lax.fori_loop(0, NB2, scale, 0)
        pltpu.sync_copy(z_hbm, acc_sh.at[pl.ds(s * RPT, RPT)])
        plsc.subcore_barrier()

        def body(j, _):
            pltpu.sync_copy(x1_hbm.at[src_v.at[j]], buf_v)
            pltpu.sync_copy(buf_v, acc_sh.at[dst_v.at[j]], add=True)
            return 0
        lax.fori_loop(0, NB2, body, 0)
        plsc.subcore_barrier()
        rows = pl.ds(s * RPT, RPT)
        pltpu.sync_copy(acc_sh.at[rows], out_hbm.at[g, rows])
        return 0
    lax.fori_loop(0, GPC, panel, 0)


@functools.lru_cache(maxsize=None)
def _agg1_kernel():
    return pl.kernel(
        _agg1_body,
        out_type=jax.ShapeDtypeStruct((NC, NPAD, 128), jnp.float32),
        mesh=_sc_mesh(),
        scratch_types=[
            pltpu.VMEM((NB1, 128), jnp.int32),
            pltpu.VMEM((NB1, 128), jnp.int32),
            pltpu.VMEM((128, 128), jnp.float32),
            pltpu.VMEM_SHARED((NPAD, 128), jnp.float32),
        ],
    )


@functools.lru_cache(maxsize=None)
def _agg2_kernel():
    return pl.kernel(
        _agg2_body,
        out_type=jax.ShapeDtypeStruct((G, NPAD, 128), jnp.float32),
        mesh=_sc_mesh(),
        scratch_types=[
            pltpu.VMEM((NB2, 128), jnp.int32),
            pltpu.VMEM((NB2, 128), jnp.int32),
            pltpu.VMEM((128, 128), jnp.float32),
            pltpu.VMEM_SHARED((NPAD, 128), jnp.float32),
        ],
    )


def _agg1(x0aug, e1s, e1d):
    z = jnp.zeros((RPT, 128), jnp.float32)
    return _agg1_kernel()(x0aug, e1s, e1d, z)


def _agg2(x1r, e2s, e2d):
    z = jnp.zeros((RPT, 128), jnp.float32)
    return _agg2_kernel()(x1r, e2s, e2d, z)


def _dense1_body(a_ref, x_ref, wl_ref, wr_ref, bl_ref, g_ref, b_ref, o_ref):
    a = a_ref[0] + a_ref[1]
    cnt = jnp.maximum(a[:, 48:49], 1.0)
    agg = a[:, :BT] / cnt
    x0 = x_ref[:, :BT]
    h = (agg[:, :, None] * wl_ref[0][None, None, :]
         + x0[:, :, None] * wr_ref[0][None, None, :]
         + bl_ref[0][None, None, :])
    mu = jnp.mean(h, axis=-1, keepdims=True)
    var = jnp.mean((h - mu) ** 2, axis=-1, keepdims=True)
    x1 = (h - mu) * lax.rsqrt(var + 1e-5) * g_ref[0] + b_ref[0]
    o_ref[...] = jnp.maximum(x1, 0.0)


def _dense2_body(agg_ref, x1_ref, a_ref, wl_ref, wr_ref, bl_ref, g_ref,
                 b_ref, o_ref):
    i = pl.program_id(0)
    bn = x1_ref.shape[0]
    a = a_ref[0] + a_ref[1]
    inv = 1.0 / jnp.maximum(a[:, 48:49], 1.0)
    agg = agg_ref[...]
    x1 = jnp.swapaxes(x1_ref[...], 0, 1)
    h = (jnp.einsum('gnk,kj->gnj', agg, wl_ref[...],
                    preferred_element_type=jnp.float32) * inv[None, :, :]
         + jnp.einsum('gnk,kj->gnj', x1, wr_ref[...],
                      preferred_element_type=jnp.float32)
         + bl_ref[0][None, None, :])
    mul = jnp.mean(h[:, :, :HG], axis=-1, keepdims=True)
    mur = jnp.mean(h[:, :, HG:], axis=-1, keepdims=True)
    mu = jnp.concatenate([jnp.broadcast_to(mul, h[:, :, :HG].shape),
                          jnp.broadcast_to(mur, h[:, :, :HG].shape)], axis=-1)
    d = h - mu
    varl = jnp.mean(d[:, :, :HG] ** 2, axis=-1, keepdims=True)
    varr = jnp.mean(d[:, :, HG:] ** 2, axis=-1, keepdims=True)
    var = jnp.concatenate([jnp.broadcast_to(varl, d[:, :, :HG].shape),
                           jnp.broadcast_to(varr, d[:, :, :HG].shape)],
                          axis=-1)
    x2 = d * lax.rsqrt(var + 1e-5) * g_ref[0] + b_ref[0]
    x2 = jnp.maximum(x2, 0.0)
    node = lax.broadcasted_iota(jnp.int32, (1, bn, 1), 1) + i * bn
    x2 = jnp.where(node < N, x2, 0.0)
    part = jnp.sum(x2, axis=1)

    @pl.when(i == 0)
    def _():
        o_ref[...] = jnp.zeros_like(o_ref)
    o_ref[...] += part


def _gru_body(h_ref, wih_ref, whh_ref, bih_ref, bhh_ref, hw_ref, hb_ref,
              o_ref):
    Hm = h_ref[...] * (1.0 / N)
    h = jnp.zeros((B, HT), jnp.float32)
    for t in range(T):
        xt = Hm[t * B:(t + 1) * B]
        gi = jnp.dot(xt, wih_ref[...],
                     preferred_element_type=jnp.float32) + bih_ref[0]
        gh = jnp.dot(h, whh_ref[...],
                     preferred_element_type=jnp.float32) + bhh_ref[0]
        r = jax.nn.sigmoid(gi[:, :HT] + gh[:, :HT])
        z = jax.nn.sigmoid(gi[:, HT:2 * HT] + gh[:, HT:2 * HT])
        n = jnp.tanh(gi[:, 2 * HT:] + r * gh[:, 2 * HT:])
        h = (1.0 - z) * n + z * h
    y = jnp.sum(h * hw_ref[...], axis=1, keepdims=True) + hb_ref[0]
    o_ref[...] = jnp.broadcast_to(y, (B, HT))


def _pad_edges(e, per, width):
    n_chunks = e.shape[0] // per
    e = e.reshape(n_chunks, per)
    pad = width * 128 - per
    e = jnp.concatenate(
        [e, jnp.full((n_chunks, pad), DUMMY, jnp.int32)], axis=1)
    return e.reshape(n_chunks, width, 128)


def kernel(x_seq, edge_index, l1_wl, l1_bl, l1_wr, ln1_g, ln1_b, l2_wl,
           l2_bl, l2_wr, ln2_g, ln2_b, gru_w_ih, gru_w_hh, gru_b_ih,
           gru_b_hh, head_w, head_b):
    f32 = jnp.float32
    x0 = x_seq[..., 0].transpose(1, 0, 2).reshape(BT, N)
    x0aug = jnp.zeros((NPAD, 128), f32)
    x0aug = x0aug.at[:N, :BT].set(x0.T)
    x0aug = x0aug.at[:N, BT].set(1.0)
    src, dst = edge_index[0], edge_index[1]
    e1s = _pad_edges(src, E // (NC * NS), NB1).reshape(NC, NS, NB1, 128)
    e1d = _pad_edges(dst, E // (NC * NS), NB1).reshape(NC, NS, NB1, 128)
    e2s = _pad_edges(src, E // NS, NB2)
    e2d = _pad_edges(dst, E // NS, NB2)
    wl1 = l1_wl.T.astype(f32)
    wr1 = l1_wr.T.astype(f32)
    row = lambda v: v.reshape(1, -1).astype(f32)
    wihT = gru_w_ih.T.astype(f32)
    whhT = gru_w_hh.T.astype(f32)

    acc1 = _agg1(x0aug, e1s, e1d)

    BN = 320
    x1 = pl.pallas_call(
        _dense1_body,
        grid=(NPAD // BN,),
        in_specs=[
            pl.BlockSpec((NC, BN, 128), lambda i: (0, i, 0)),
            pl.BlockSpec((BN, 128), lambda i: (i, 0)),
            pl.BlockSpec((1, 64), lambda i: (0, 0)),
            pl.BlockSpec((1, 64), lambda i: (0, 0)),
            pl.BlockSpec((1, 64), lambda i: (0, 0)),
            pl.BlockSpec((1, 64), lambda i: (0, 0)),
            pl.BlockSpec((1, 64), lambda i: (0, 0)),
        ],
        out_specs=pl.BlockSpec((BN, BT, HG), lambda i: (i, 0, 0)),
        out_shape=jax.ShapeDtypeStruct((NPAD, BT, HG), f32),
    )(acc1, x0aug, wl1, wr1, row(l1_bl), row(ln1_g), row(ln1_b))

    x1r = x1.reshape(NPAD * G, 128)
    agg2 = _agg2(x1r, e2s, e2d)

    BN2 = 640
    bd = jnp.zeros((128, 128), f32)
    wl2bd = bd.at[:HG, :HG].set(l2_wl.T).at[HG:, HG:].set(l2_wl.T)
    wr2bd = bd.at[:HG, :HG].set(l2_wr.T).at[HG:, HG:].set(l2_wr.T)
    two = lambda v: jnp.concatenate([v, v]).reshape(1, 128).astype(f32)
    x1p = x1.reshape(NPAD, G, 128)
    hsum = pl.pallas_call(
        _dense2_body,
        grid=(NPAD // BN2,),
        in_specs=[
            pl.BlockSpec((G, BN2, 128), lambda i: (0, i, 0)),
            pl.BlockSpec((BN2, G, 128), lambda i: (i, 0, 0)),
            pl.BlockSpec((NC, BN2, 128), lambda i: (0, i, 0)),
            pl.BlockSpec((128, 128), lambda i: (0, 0)),
            pl.BlockSpec((128, 128), lambda i: (0, 0)),
            pl.BlockSpec((1, 128), lambda i: (0, 0)),
            pl.BlockSpec((1, 128), lambda i: (0, 0)),
            pl.BlockSpec((1, 128), lambda i: (0, 0)),
        ],
        out_specs=pl.BlockSpec((G, 128), lambda i: (0, 0)),
        out_shape=jax.ShapeDtypeStruct((G, 128), f32),
    )(agg2, x1p, acc1, wl2bd, wr2bd, two(l2_bl), two(ln2_g), two(ln2_b))
    hsum = hsum.reshape(BT, HG)

    yb = pl.pallas_call(
        _gru_body,
        in_specs=[
            pl.BlockSpec((BT, HG), lambda: (0, 0)),
            pl.BlockSpec((HG, 3 * HT), lambda: (0, 0)),
            pl.BlockSpec((HT, 3 * HT), lambda: (0, 0)),
            pl.BlockSpec((1, 3 * HT), lambda: (0, 0)),
            pl.BlockSpec((1, 3 * HT), lambda: (0, 0)),
            pl.BlockSpec((1, HT), lambda: (0, 0)),
            pl.BlockSpec((1, 1), lambda: (0, 0)),
        ],
        out_specs=pl.BlockSpec((B, HT), lambda: (0, 0)),
        out_shape=jax.ShapeDtypeStruct((B, HT), f32),
    )(hsum, wihT, whhT, row(gru_b_ih), row(gru_b_hh), head_w.astype(f32),
      head_b.reshape(1, 1).astype(f32))
    return yb[:, 0]
